# baseline (speedup 1.0000x reference)
import jax
import jax.numpy as jnp
from jax import lax
from jax.experimental import pallas as pl
from jax.experimental.pallas import tpu as pltpu
from jax.experimental.pallas import tpu_sc as plsc

EMBED_DIM = 128
BATCH = 16384

NC = 2
NS = 16
NW = NC * NS
BPW = BATCH // NW
CHUNK = 64
NCH = BPW // CHUNK
NBUF = 14


def _body(idx_hbm, p_tab, g_tab, out_p, out_g,
          idx_v, rows_v, gsems, wsems):
    wid = lax.axis_index("s") * NC + lax.axis_index("c")
    base = wid * BPW

    pltpu.sync_copy(idx_hbm.at[wid], idx_v)

    tabs = [p_tab, g_tab]
    outs = [out_p, out_g]

    def gather(t, j, buf):
        return pltpu.async_copy(tabs[t].at[idx_v.at[t, j]], rows_v.at[buf],
                                gsems.at[buf])

    def writeback(t, j, buf):
        return pltpu.async_copy(rows_v.at[buf],
                                outs[t].at[pl.ds(base + j * CHUNK, CHUNK)],
                                wsems.at[buf])

    work = [(t, j) for t in range(2) for j in range(NCH)]
    n = len(work)

    hg = [None] * NBUF
    hw = [None] * NBUF
    for i in range(min(NBUF, n)):
        hg[i] = gather(*work[i], i)

    for i in range(n):
        buf = i % NBUF
        hg[buf].wait()
        hw[buf] = writeback(*work[i], buf)
        nxt = i + NBUF
        if nxt < n:
            hw[buf].wait()
            hg[buf] = gather(*work[nxt], buf)

    for i in range(max(0, n - NBUF), n):
        hw[i % NBUF].wait()


def kernel(edges_protein, edges_gene, protein_table, gene_table):
    idx_p = edges_protein.astype(jnp.int32).reshape(NW, 1, NCH, CHUNK)
    idx_g = edges_gene.astype(jnp.int32).reshape(NW, 1, NCH, CHUNK)
    idx_all = jnp.concatenate([idx_p, idx_g], axis=1)

    mesh = plsc.VectorSubcoreMesh(core_axis_name="c", subcore_axis_name="s")
    run = pl.kernel(
        _body,
        out_type=(
            jax.ShapeDtypeStruct((BATCH, EMBED_DIM), jnp.float32),
            jax.ShapeDtypeStruct((BATCH, EMBED_DIM), jnp.float32),
        ),
        mesh=mesh,
        scratch_types=[
            pltpu.VMEM((2, NCH, CHUNK), jnp.int32),
            pltpu.VMEM((NBUF, CHUNK, EMBED_DIM), jnp.float32),
            pltpu.SemaphoreType.DMA((NBUF,)),
            pltpu.SemaphoreType.DMA((NBUF,)),
        ],
    )
    return run(idx_all, protein_table, gene_table)


# speedup vs baseline: 1.0172x; 1.0172x over previous
import jax
import jax.numpy as jnp
from jax import lax
from jax.experimental import pallas as pl
from jax.experimental.pallas import tpu as pltpu
from jax.experimental.pallas import tpu_sc as plsc

EMBED_DIM = 128
BATCH = 16384

NC = 2
NS = 16
NW = NC * NS
BPW = BATCH // NW
CHUNK = 128
NCH = BPW // CHUNK
NBUF = 7


def _body(idx_hbm, p_tab, g_tab, out_p, out_g,
          idx_v, rows_v, gsems, wsems):
    wid = lax.axis_index("s") * NC + lax.axis_index("c")
    base = wid * BPW

    pltpu.sync_copy(idx_hbm.at[wid], idx_v)

    tabs = [p_tab, g_tab]
    outs = [out_p, out_g]

    def gather(t, j, buf):
        return pltpu.async_copy(tabs[t].at[idx_v.at[t, j]], rows_v.at[buf],
                                gsems.at[buf])

    def writeback(t, j, buf):
        return pltpu.async_copy(rows_v.at[buf],
                                outs[t].at[pl.ds(base + j * CHUNK, CHUNK)],
                                wsems.at[buf])

    work = [(t, j) for j in range(NCH) for t in range(2)]
    n = len(work)

    hg = [None] * NBUF
    hw = [None] * NBUF
    for i in range(min(NBUF, n)):
        hg[i] = gather(*work[i], i)

    for i in range(n):
        buf = i % NBUF
        hg[buf].wait()
        hw[buf] = writeback(*work[i], buf)
        nxt = i + NBUF
        if nxt < n:
            hw[buf].wait()
            hg[buf] = gather(*work[nxt], buf)

    for i in range(max(0, n - NBUF), n):
        hw[i % NBUF].wait()


def kernel(edges_protein, edges_gene, protein_table, gene_table):
    idx_p = edges_protein.astype(jnp.int32).reshape(NW, 1, NCH, CHUNK)
    idx_g = edges_gene.astype(jnp.int32).reshape(NW, 1, NCH, CHUNK)
    idx_all = jnp.concatenate([idx_p, idx_g], axis=1)

    mesh = plsc.VectorSubcoreMesh(core_axis_name="c", subcore_axis_name="s")
    run = pl.kernel(
        _body,
        out_type=(
            jax.ShapeDtypeStruct((BATCH, EMBED_DIM), jnp.float32),
            jax.ShapeDtypeStruct((BATCH, EMBED_DIM), jnp.float32),
        ),
        mesh=mesh,
        scratch_types=[
            pltpu.VMEM((2, NCH, CHUNK), jnp.int32),
            pltpu.VMEM((NBUF, CHUNK, EMBED_DIM), jnp.float32),
            pltpu.SemaphoreType.DMA((NBUF,)),
            pltpu.SemaphoreType.DMA((NBUF,)),
        ],
    )
    return run(idx_all, protein_table, gene_table)


# speedup vs baseline: 1.0229x; 1.0055x over previous
import jax
import jax.numpy as jnp
from jax import lax
from jax.experimental import pallas as pl
from jax.experimental.pallas import tpu as pltpu
from jax.experimental.pallas import tpu_sc as plsc

EMBED_DIM = 128
BATCH = 16384

NC = 2
NS = 16
NW = NC * NS
BPW = BATCH // NW
CHUNK = 128
NCH = BPW // CHUNK
NBUF = 7


def _body(idx_hbm, p_tab, g_tab, out_p, out_g,
          idx_v, rows_v, gsems, wsems):
    wid = lax.axis_index("s") * NC + lax.axis_index("c")
    base = wid * BPW

    pltpu.sync_copy(idx_hbm.at[wid], idx_v)

    tabs = [p_tab, g_tab]
    outs = [out_p, out_g]

    def gather(t, j, buf):
        return pltpu.async_copy(tabs[t].at[idx_v.at[t, j]], rows_v.at[buf],
                                gsems.at[buf])

    def writeback(t, j, buf):
        return pltpu.async_copy(rows_v.at[buf],
                                outs[t].at[pl.ds(base + j * CHUNK, CHUNK)],
                                wsems.at[buf])

    work = [(t, j) for t in range(2) for j in range(NCH)]
    n = len(work)

    hg = [None] * NBUF
    hw = [None] * NBUF
    for i in range(min(NBUF, n)):
        hg[i] = gather(*work[i], i)

    for i in range(n):
        buf = i % NBUF
        hg[buf].wait()
        hw[buf] = writeback(*work[i], buf)
        nxt = i + NBUF
        if nxt < n:
            hw[buf].wait()
            hg[buf] = gather(*work[nxt], buf)

    for i in range(max(0, n - NBUF), n):
        hw[i % NBUF].wait()


def kernel(edges_protein, edges_gene, protein_table, gene_table):
    idx_p = edges_protein.astype(jnp.int32).reshape(NW, 1, NCH, CHUNK)
    idx_g = edges_gene.astype(jnp.int32).reshape(NW, 1, NCH, CHUNK)
    idx_all = jnp.concatenate([idx_p, idx_g], axis=1)

    mesh = plsc.VectorSubcoreMesh(core_axis_name="c", subcore_axis_name="s")
    run = pl.kernel(
        _body,
        out_type=(
            jax.ShapeDtypeStruct((BATCH, EMBED_DIM), jnp.float32),
            jax.ShapeDtypeStruct((BATCH, EMBED_DIM), jnp.float32),
        ),
        mesh=mesh,
        scratch_types=[
            pltpu.VMEM((2, NCH, CHUNK), jnp.int32),
            pltpu.VMEM((NBUF, CHUNK, EMBED_DIM), jnp.float32),
            pltpu.SemaphoreType.DMA((NBUF,)),
            pltpu.SemaphoreType.DMA((NBUF,)),
        ],
    )
    return run(idx_all, protein_table, gene_table)
